# baseline (speedup 1.0000x reference)
import functools

import jax
import jax.numpy as jnp
from jax import lax
from jax.experimental import pallas as pl
from jax.experimental.pallas import tpu as pltpu
from jax.experimental.pallas import tpu_sc as plsc

VOCAB = 1000000
EMBED_DIMS = 32
BATCH = 16384
EPS = 1e-05

_NC = 2
_NS = 16
_NW = _NC * _NS
_BPW = BATCH // _NW
_CHUNK = 128
_NCHUNK = _BPW // _CHUNK


def _sc_gather_kernel(table_hbm, left_hbm, right_hbm, out_hbm,
                      lidx_v, ridx_v, u_v, v_v, sem):
    wid = lax.axis_index("s") * _NC + lax.axis_index("c")
    base = wid * _BPW

    pltpu.sync_copy(left_hbm.at[wid], lidx_v)
    pltpu.sync_copy(right_hbm.at[wid], ridx_v)

    copies = []
    for j in range(_NCHUNK):
        sl = pl.ds(j * _CHUNK, _CHUNK)
        copies.append(pltpu.async_copy(table_hbm.at[lidx_v.at[j]], u_v.at[sl], sem))
        copies.append(pltpu.async_copy(table_hbm.at[ridx_v.at[j]], v_v.at[sl], sem))
    for c in copies:
        c.wait()

    pltpu.sync_copy(u_v, out_hbm.at[0, pl.ds(base, _BPW)])
    pltpu.sync_copy(v_v, out_hbm.at[1, pl.ds(base, _BPW)])


@jax.jit
def _sc_gather(table, left3, right3):
    mesh = plsc.VectorSubcoreMesh(core_axis_name="c", subcore_axis_name="s")
    kfn = functools.partial(
        pl.kernel,
        mesh=mesh,
        out_type=jax.ShapeDtypeStruct((2, BATCH, EMBED_DIMS), jnp.float32),
        scratch_types=[
            pltpu.VMEM((_NCHUNK, _CHUNK), jnp.int32),
            pltpu.VMEM((_NCHUNK, _CHUNK), jnp.int32),
            pltpu.VMEM((_BPW, EMBED_DIMS), jnp.float32),
            pltpu.VMEM((_BPW, EMBED_DIMS), jnp.float32),
            pltpu.SemaphoreType.DMA,
        ],
        compiler_params=pltpu.CompilerParams(use_tc_tiling_on_sc=False),
    )(_sc_gather_kernel)
    return kfn(table, left3, right3)


def _tc_dist_kernel(rows_ref, o_ref):
    u = rows_ref[0]
    v = rows_ref[1]
    uu = jnp.sum(u * u, axis=-1)
    vv = jnp.sum(v * v, axis=-1)
    uv = jnp.sum(u * v, axis=-1)
    alpha = 1.0 - uu
    alpha = jnp.where(alpha <= 0.0, EPS, alpha)
    beta = 1.0 - vv
    beta = jnp.where(beta <= 0.0, EPS, beta)
    gamma = 1.0 + 2.0 * (uu - 2.0 * uv + vv) / alpha / beta
    gamma = jnp.where(gamma < 1.0, 1.0, gamma)
    o_ref[...] = jnp.log(gamma + jnp.sqrt(gamma * gamma - 1.0))


@jax.jit
def _tc_dist(rows):
    return pl.pallas_call(
        _tc_dist_kernel,
        out_shape=jax.ShapeDtypeStruct((BATCH,), jnp.float32),
    )(rows)


def kernel(left_idx, right_idx, table):
    left3 = left_idx.astype(jnp.int32).reshape(_NW, _NCHUNK, _CHUNK)
    right3 = right_idx.astype(jnp.int32).reshape(_NW, _NCHUNK, _CHUNK)
    rows = _sc_gather(table, left3, right3)
    return _tc_dist(rows)
